# baseline (speedup 1.0000x reference)
import jax
import jax.numpy as jnp
from jax.experimental import pallas as pl
from jax.experimental.pallas import tpu as pltpu

R = 128
C = 32768
BR = 8
NB = R // BR
BIG = 2**30


AM_BC = 8192
AM_NB = C // AM_BC


def _argmax_body(x_ref, idx_ref, run_max, run_idx):
    j = pl.program_id(0)
    x = x_ref[...]
    m = jnp.max(x, axis=1, keepdims=True)
    loc = jnp.argmax(x, axis=1).astype(jnp.int32).reshape(R, 1) + j * AM_BC

    @pl.when(j == 0)
    def _():
        run_max[...] = m
        run_idx[...] = loc

    @pl.when(j > 0)
    def _():
        better = m > run_max[...]
        run_idx[...] = jnp.where(better, loc, run_idx[...])
        run_max[...] = jnp.maximum(m, run_max[...])

    @pl.when(j == AM_NB - 1)
    def _():
        idx_ref[...] = run_idx[...]


OH_BR = 128
OH_BC = 16384
OH_NB = (R // OH_BR) * (C // OH_BC)
OH_NCB = C // OH_BC


def _onehot_body(idx_ref, out_ref):
    j = pl.program_id(0)
    col = jax.lax.broadcasted_iota(jnp.int32, (OH_BR, OH_BC), 1) + (j % OH_NCB) * OH_BC
    out_ref[...] = (col == idx_ref[...]).astype(jnp.float32)


@jax.jit
def kernel(Xsoft, P):
    del P
    idx = pl.pallas_call(
        _argmax_body,
        grid=(AM_NB,),
        in_specs=[pl.BlockSpec((R, AM_BC), lambda j: (0, j))],
        out_specs=pl.BlockSpec((R, 1), lambda j: (0, 0)),
        out_shape=jax.ShapeDtypeStruct((R, 1), jnp.int32),
        scratch_shapes=[
            pltpu.VMEM((R, 1), jnp.float32),
            pltpu.VMEM((R, 1), jnp.int32),
        ],
    )(Xsoft)

    out = pl.pallas_call(
        _onehot_body,
        grid=(OH_NB,),
        in_specs=[pl.BlockSpec((OH_BR, 1), lambda j: (j // OH_NCB, 0))],
        out_specs=pl.BlockSpec((OH_BR, OH_BC), lambda j: (j // OH_NCB, j % OH_NCB)),
        out_shape=jax.ShapeDtypeStruct((R, C), jnp.float32),
    )(idx)
    return out
